# Initial kernel scaffold; baseline (speedup 1.0000x reference)
import functools

import jax
import jax.numpy as jnp
from jax import lax
from jax.experimental import pallas as pl
from jax.experimental.pallas import tpu as pltpu
from jax.experimental.pallas import tpu_sc as plsc

T = 4096
D = 768
H = 768
E = 64
G = 32
GB = T // G
BLK = 128
NW = 32
WCHUNK = T // NW


def _router_body(x_ref, wr_ref, br_ref, pos_ref, gate16_ref, meta_ref,
                 rankw_ref, totg_ref):
    x = x_ref[...]
    logits = jnp.dot(x, wr_ref[...], preferred_element_type=jnp.float32)
    logits = logits + br_ref[...].reshape(1, E)
    m = jnp.max(logits, axis=1, keepdims=True)
    el = jnp.exp(logits - m)
    s = jnp.sum(el, axis=1, keepdims=True)
    p = el / s
    pmax = jnp.max(p, axis=1, keepdims=True)
    lane = lax.broadcasted_iota(jnp.int32, (T, E), 1)
    idx = jnp.min(jnp.where(p >= pmax, lane, E), axis=1, keepdims=True)
    onehot = (lane == idx).astype(jnp.float32)
    gate = pmax / (pmax + 1e-6)
    gate16_ref[...] = jnp.broadcast_to(gate, (T, 16))

    tri = (lax.broadcasted_iota(jnp.int32, (GB, GB), 0)
           >= lax.broadcasted_iota(jnp.int32, (GB, GB), 1)).astype(jnp.float32)

    def g1(g, _):
        oh = lax.dynamic_slice(onehot, (g * GB, 0), (GB, E))
        c = jnp.dot(tri, oh, preferred_element_type=jnp.float32)
        rankw_ref[pl.ds(g * GB, GB), :] = jnp.sum(c * oh, axis=1, keepdims=True)
        totg_ref[pl.ds(g, 1), :] = jnp.sum(oh, axis=0, keepdims=True)
        return 0

    lax.fori_loop(0, G, g1, 0)

    totg = totg_ref[...]
    sl = (lax.broadcasted_iota(jnp.int32, (G, G), 0)
          > lax.broadcasted_iota(jnp.int32, (G, G), 1)).astype(jnp.float32)
    goff = jnp.dot(sl, totg, preferred_element_type=jnp.float32)
    counts = jnp.sum(totg, axis=0, keepdims=True)
    su = (lax.broadcasted_iota(jnp.int32, (E, E), 0)
          < lax.broadcasted_iota(jnp.int32, (E, E), 1)).astype(jnp.float32)
    offsets = jnp.dot(counts, su, preferred_element_type=jnp.float32)
    combined = offsets + goff

    def g2(g, _):
        oh = lax.dynamic_slice(onehot, (g * GB, 0), (GB, E))
        base = jnp.sum(oh * lax.dynamic_slice(combined, (g, 0), (1, E)),
                       axis=1, keepdims=True)
        rw = rankw_ref[pl.ds(g * GB, GB), :]
        pos_ref[pl.ds(g * GB, GB), :] = (base + rw - 1.0).astype(jnp.int32)
        return 0

    lax.fori_loop(0, G, g2, 0)

    meta_ref[...] = jnp.concatenate(
        [offsets, counts], axis=1).astype(jnp.int32)


def _router(x, wr, br):
    return pl.pallas_call(
        _router_body,
        out_shape=(
            jax.ShapeDtypeStruct((T, 1), jnp.int32),
            jax.ShapeDtypeStruct((T, 16), jnp.float32),
            jax.ShapeDtypeStruct((1, 2 * E), jnp.int32),
        ),
        scratch_shapes=[
            pltpu.VMEM((T, 1), jnp.float32),
            pltpu.VMEM((G, E), jnp.float32),
        ],
    )(x, wr, br)


_SC_MESH = plsc.VectorSubcoreMesh(core_axis_name="c", subcore_axis_name="s")


def _dispatch(x, pos, gate16):
    @functools.partial(
        pl.kernel,
        mesh=_SC_MESH,
        out_type=(
            jax.ShapeDtypeStruct((T, D), jnp.float32),
            jax.ShapeDtypeStruct((T, 16), jnp.float32),
        ),
        scratch_types=[
            pltpu.VMEM((WCHUNK,), jnp.int32),
            pltpu.VMEM((WCHUNK, D), jnp.float32),
            pltpu.VMEM((WCHUNK, 16), jnp.float32),
            pltpu.SemaphoreType.DMA,
        ],
    )
    def disp(x_hbm, pos_hbm, g_hbm, xs_hbm, gs_hbm, idx_v, rows_v, g_v, sem):
        wid = lax.axis_index("s") * 2 + lax.axis_index("c")
        base = wid * WCHUNK
        pltpu.sync_copy(pos_hbm.at[pl.ds(base, WCHUNK)], idx_v)
        pltpu.sync_copy(x_hbm.at[pl.ds(base, WCHUNK)], rows_v)
        pltpu.sync_copy(g_hbm.at[pl.ds(base, WCHUNK)], g_v)
        pltpu.async_copy(rows_v, xs_hbm.at[idx_v], sem).wait()
        pltpu.async_copy(g_v, gs_hbm.at[idx_v], sem).wait()

    return disp(x, pos, gate16)


def _unsort(ys, pos):
    @functools.partial(
        pl.kernel,
        mesh=_SC_MESH,
        out_type=jax.ShapeDtypeStruct((T, D), jnp.float32),
        scratch_types=[
            pltpu.VMEM((WCHUNK,), jnp.int32),
            pltpu.VMEM((WCHUNK, D), jnp.float32),
            pltpu.SemaphoreType.DMA,
        ],
    )
    def gath(ys_hbm, pos_hbm, out_hbm, idx_v, rows_v, sem):
        wid = lax.axis_index("s") * 2 + lax.axis_index("c")
        base = wid * WCHUNK
        pltpu.sync_copy(pos_hbm.at[pl.ds(base, WCHUNK)], idx_v)
        pltpu.async_copy(ys_hbm.at[idx_v], rows_v, sem).wait()
        pltpu.sync_copy(rows_v, out_hbm.at[pl.ds(base, WCHUNK)])

    return gath(ys, pos)


def _gemm_body(meta_ref, xs_ref, gs_ref, w1_ref, b1_ref, w2_ref, b2_ref,
               w3_ref, b3_ref, out_ref):
    e = pl.program_id(0)
    off = meta_ref[e]
    cnt = meta_ref[E + e]
    w1 = w1_ref[0]
    w2 = w2_ref[0]
    w3 = w3_ref[0]
    b1 = b1_ref[0].reshape(1, H)
    b2 = b2_ref[0].reshape(1, H)
    b3 = b3_ref[0].reshape(1, D)
    nt = (cnt + BLK - 1) // BLK

    def chunk(i, _):
        start = jnp.minimum(off + i * BLK, T - BLK)
        rows = xs_ref[pl.ds(start, BLK), :]
        h = jnp.maximum(
            jnp.dot(rows, w1, preferred_element_type=jnp.float32) + b1, 0.0)
        h = jnp.maximum(
            jnp.dot(h, w2, preferred_element_type=jnp.float32) + b2, 0.0)
        o = jnp.dot(h, w3, preferred_element_type=jnp.float32) + b3
        o = o * gs_ref[pl.ds(start, BLK), 0:1]
        rowid = start + lax.broadcasted_iota(jnp.int32, (BLK, 1), 0)
        mask = (rowid >= off) & (rowid < off + cnt)
        cur = out_ref[pl.ds(start, BLK), :]
        out_ref[pl.ds(start, BLK), :] = jnp.where(mask, o, cur)
        return 0

    lax.fori_loop(0, nt, chunk, 0)


def _gemm(meta, xs, gs, w1, b1, w2, b2, w3, b3):
    grid_spec = pltpu.PrefetchScalarGridSpec(
        num_scalar_prefetch=1,
        grid=(E,),
        in_specs=[
            pl.BlockSpec((T, D), lambda e, m: (0, 0)),
            pl.BlockSpec((T, 16), lambda e, m: (0, 0)),
            pl.BlockSpec((1, D, H), lambda e, m: (e, 0, 0)),
            pl.BlockSpec((1, H), lambda e, m: (e, 0)),
            pl.BlockSpec((1, H, H), lambda e, m: (e, 0, 0)),
            pl.BlockSpec((1, H), lambda e, m: (e, 0)),
            pl.BlockSpec((1, H, D), lambda e, m: (e, 0, 0)),
            pl.BlockSpec((1, D), lambda e, m: (e, 0)),
        ],
        out_specs=pl.BlockSpec((T, D), lambda e, m: (0, 0)),
    )
    return pl.pallas_call(
        _gemm_body,
        grid_spec=grid_spec,
        out_shape=jax.ShapeDtypeStruct((T, D), jnp.float32),
        compiler_params=pltpu.CompilerParams(
            dimension_semantics=("arbitrary",),
        ),
    )(meta, xs, gs, w1, b1, w2, b2, w3, b3)


def kernel(x, Wr, br, W1, b1, W2, b2, W3, b3):
    pos2d, gate16, meta2d = _router(x, Wr, br)
    pos = pos2d.reshape(T)
    meta = meta2d.reshape(2 * E)
    xs, gs = _dispatch(x, pos, gate16)
    ys = _gemm(meta, xs, gs, W1, b1, W2, b2, W3, b3)
    return _unsort(ys, pos)


# speedup vs baseline: 10.5446x; 10.5446x over previous
import functools

import jax
import jax.numpy as jnp
from jax import lax
from jax.experimental import pallas as pl
from jax.experimental.pallas import tpu as pltpu
from jax.experimental.pallas import tpu_sc as plsc

T = 4096
D = 768
H = 768
E = 64
G = 32
GB = T // G
BLK = 128
NW = 32
WCHUNK = T // NW


def _router_body(x_ref, wr_ref, br_ref, pos_ref, gate16_ref, meta_ref,
                 rankw_ref, totg_ref, oh_ref):
    x = x_ref[...]
    logits = jnp.dot(x, wr_ref[...], preferred_element_type=jnp.float32)
    logits = logits + br_ref[...].reshape(1, E)
    m = jnp.max(logits, axis=1, keepdims=True)
    el = jnp.exp(logits - m)
    s = jnp.sum(el, axis=1, keepdims=True)
    p = el / s
    pmax = jnp.max(p, axis=1, keepdims=True)
    lane = lax.broadcasted_iota(jnp.int32, (T, E), 1)
    idx = jnp.min(jnp.where(p >= pmax, lane, E), axis=1, keepdims=True)
    oh_ref[...] = (lane == idx).astype(jnp.float32)
    gate = pmax / (pmax + 1e-6)
    gate16_ref[...] = jnp.broadcast_to(gate, (T, 128))

    tri = (lax.broadcasted_iota(jnp.int32, (GB, GB), 0)
           >= lax.broadcasted_iota(jnp.int32, (GB, GB), 1)).astype(jnp.float32)

    def g1(g, _):
        oh = oh_ref[pl.ds(g * GB, GB), :]
        c = jnp.dot(tri, oh, preferred_element_type=jnp.float32)
        rankw_ref[pl.ds(g * GB, GB), :] = jnp.sum(c * oh, axis=1, keepdims=True)
        totg_ref[pl.ds(g, 1), :] = jnp.sum(oh, axis=0, keepdims=True)
        return 0

    lax.fori_loop(0, G, g1, 0)

    totg = totg_ref[...]
    sl = (lax.broadcasted_iota(jnp.int32, (G, G), 0)
          > lax.broadcasted_iota(jnp.int32, (G, G), 1)).astype(jnp.float32)
    goff = jnp.dot(sl, totg, preferred_element_type=jnp.float32)
    counts = jnp.sum(totg, axis=0, keepdims=True)
    su = (lax.broadcasted_iota(jnp.int32, (E, E), 0)
          < lax.broadcasted_iota(jnp.int32, (E, E), 1)).astype(jnp.float32)
    offsets = jnp.dot(counts, su, preferred_element_type=jnp.float32)
    totg_ref[...] = offsets + goff

    def g2(g, _):
        oh = oh_ref[pl.ds(g * GB, GB), :]
        base = jnp.sum(oh * totg_ref[pl.ds(g, 1), :],
                       axis=1, keepdims=True)
        rw = rankw_ref[pl.ds(g * GB, GB), :]
        pos_ref[pl.ds(g * GB, GB), :] = (base + rw - 1.0).astype(jnp.int32)
        return 0

    lax.fori_loop(0, G, g2, 0)

    meta_ref[...] = jnp.concatenate(
        [offsets, counts], axis=1).astype(jnp.int32)


def _router(x, wr, br):
    return pl.pallas_call(
        _router_body,
        out_shape=(
            jax.ShapeDtypeStruct((T, 1), jnp.int32),
            jax.ShapeDtypeStruct((T, 128), jnp.float32),
            jax.ShapeDtypeStruct((1, 2 * E), jnp.int32),
        ),
        scratch_shapes=[
            pltpu.VMEM((T, 1), jnp.float32),
            pltpu.VMEM((G, E), jnp.float32),
            pltpu.VMEM((T, E), jnp.float32),
        ],
    )(x, wr, br)


_SC_MESH = plsc.VectorSubcoreMesh(core_axis_name="c", subcore_axis_name="s")


def _dispatch(x, pos, gate16):
    @functools.partial(
        pl.kernel,
        mesh=_SC_MESH,
        out_type=(
            jax.ShapeDtypeStruct((T, D), jnp.float32),
            jax.ShapeDtypeStruct((T, 128), jnp.float32),
        ),
        scratch_types=[
            pltpu.VMEM((WCHUNK,), jnp.int32),
            pltpu.VMEM((WCHUNK, D), jnp.float32),
            pltpu.VMEM((WCHUNK, 128), jnp.float32),
            pltpu.SemaphoreType.DMA,
        ],
    )
    def disp(x_hbm, pos_hbm, g_hbm, xs_hbm, gs_hbm, idx_v, rows_v, g_v, sem):
        wid = lax.axis_index("s") * 2 + lax.axis_index("c")
        base = wid * WCHUNK
        pltpu.sync_copy(pos_hbm.at[pl.ds(base, WCHUNK)], idx_v)
        pltpu.sync_copy(x_hbm.at[pl.ds(base, WCHUNK)], rows_v)
        pltpu.sync_copy(g_hbm.at[pl.ds(base, WCHUNK)], g_v)
        pltpu.async_copy(rows_v, xs_hbm.at[idx_v], sem).wait()
        pltpu.async_copy(g_v, gs_hbm.at[idx_v], sem).wait()

    return disp(x, pos, gate16)


def _unsort(ys, pos):
    @functools.partial(
        pl.kernel,
        mesh=_SC_MESH,
        out_type=jax.ShapeDtypeStruct((T, D), jnp.float32),
        scratch_types=[
            pltpu.VMEM((WCHUNK,), jnp.int32),
            pltpu.VMEM((WCHUNK, D), jnp.float32),
            pltpu.SemaphoreType.DMA,
        ],
    )
    def gath(ys_hbm, pos_hbm, out_hbm, idx_v, rows_v, sem):
        wid = lax.axis_index("s") * 2 + lax.axis_index("c")
        base = wid * WCHUNK
        pltpu.sync_copy(pos_hbm.at[pl.ds(base, WCHUNK)], idx_v)
        pltpu.async_copy(ys_hbm.at[idx_v], rows_v, sem).wait()
        pltpu.sync_copy(rows_v, out_hbm.at[pl.ds(base, WCHUNK)])

    return gath(ys, pos)


def _gemm_body(meta_ref, xs_ref, gs_ref, w1_ref, b1_ref, w2_ref, b2_ref,
               w3_ref, b3_ref, out_ref):
    e = pl.program_id(0)
    off = meta_ref[e]
    cnt = meta_ref[E + e]
    w1 = w1_ref[0]
    w2 = w2_ref[0]
    w3 = w3_ref[0]
    b1 = b1_ref[0]
    b2 = b2_ref[0]
    b3 = b3_ref[0]
    off8 = (off // 8) * 8
    d = off - off8
    nt = jnp.where(cnt > 0, (cnt + d + BLK - 1) // BLK, 0)

    def chunk(i, _):
        start = pl.multiple_of(jnp.minimum(off8 + i * BLK, T - BLK), 8)
        rows = xs_ref[pl.ds(start, BLK), :]
        h = jnp.maximum(
            jnp.dot(rows, w1, preferred_element_type=jnp.float32) + b1, 0.0)
        h = jnp.maximum(
            jnp.dot(h, w2, preferred_element_type=jnp.float32) + b2, 0.0)
        o = jnp.dot(h, w3, preferred_element_type=jnp.float32) + b3
        o = o * gs_ref[pl.ds(start, BLK), 0:1]
        rowid = start + lax.broadcasted_iota(jnp.int32, (BLK, 1), 0)
        mask = (rowid >= off) & (rowid < off + cnt)
        cur = out_ref[pl.ds(start, BLK), :]
        out_ref[pl.ds(start, BLK), :] = jnp.where(mask, o, cur)
        return 0

    lax.fori_loop(0, nt, chunk, 0)


def _gemm(meta, xs, gs, w1, b1, w2, b2, w3, b3):
    grid_spec = pltpu.PrefetchScalarGridSpec(
        num_scalar_prefetch=1,
        grid=(E,),
        in_specs=[
            pl.BlockSpec((T, D), lambda e, m: (0, 0)),
            pl.BlockSpec((T, 128), lambda e, m: (0, 0)),
            pl.BlockSpec((1, D, H), lambda e, m: (e, 0, 0)),
            pl.BlockSpec((1, 1, H), lambda e, m: (e, 0, 0)),
            pl.BlockSpec((1, H, H), lambda e, m: (e, 0, 0)),
            pl.BlockSpec((1, 1, H), lambda e, m: (e, 0, 0)),
            pl.BlockSpec((1, H, D), lambda e, m: (e, 0, 0)),
            pl.BlockSpec((1, 1, D), lambda e, m: (e, 0, 0)),
        ],
        out_specs=pl.BlockSpec((T, D), lambda e, m: (0, 0)),
    )
    return pl.pallas_call(
        _gemm_body,
        grid_spec=grid_spec,
        out_shape=jax.ShapeDtypeStruct((T, D), jnp.float32),
        compiler_params=pltpu.CompilerParams(
            dimension_semantics=("arbitrary",),
        ),
    )(meta, xs, gs, w1, b1, w2, b2, w3, b3)


def kernel(x, Wr, br, W1, b1, W2, b2, W3, b3):
    pos2d, gate16, meta2d = _router(x, Wr, br)
    pos = pos2d.reshape(T)
    meta = meta2d.reshape(2 * E)
    xs, gs = _dispatch(x, pos, gate16)
    ys = _gemm(meta, xs, gs,
               W1, b1.reshape(E, 1, H), W2, b2.reshape(E, 1, H),
               W3, b3.reshape(E, 1, D))
    return _unsort(ys, pos)


# speedup vs baseline: 10.6619x; 1.0111x over previous
import functools

import jax
import jax.numpy as jnp
from jax import lax
from jax.experimental import pallas as pl
from jax.experimental.pallas import tpu as pltpu
from jax.experimental.pallas import tpu_sc as plsc

T = 4096
D = 768
H = 768
E = 64
G = 32
GB = T // G
BLK = 128
NW = 32
WCHUNK = T // NW


def _router_body(x_ref, wr_ref, br_ref, pos_ref, gate16_ref, meta_ref,
                 rankw_ref, totg_ref, oh_ref):
    x = x_ref[...]
    logits = jnp.dot(x, wr_ref[...], preferred_element_type=jnp.float32)
    logits = logits + br_ref[...].reshape(1, E)
    m = jnp.max(logits, axis=1, keepdims=True)
    el = jnp.exp(logits - m)
    s = jnp.sum(el, axis=1, keepdims=True)
    p = el / s
    pmax = jnp.max(p, axis=1, keepdims=True)
    lane = lax.broadcasted_iota(jnp.int32, (T, E), 1)
    idx = jnp.min(jnp.where(p >= pmax, lane, E), axis=1, keepdims=True)
    oh_ref[...] = (lane == idx).astype(jnp.float32)
    gate = pmax / (pmax + 1e-6)
    gate16_ref[...] = jnp.broadcast_to(gate, (T, 128))

    tri = (lax.broadcasted_iota(jnp.int32, (GB, GB), 0)
           >= lax.broadcasted_iota(jnp.int32, (GB, GB), 1)).astype(jnp.float32)

    def g1(g, _):
        oh = oh_ref[pl.ds(g * GB, GB), :]
        c = jnp.dot(tri, oh, preferred_element_type=jnp.float32)
        rankw_ref[pl.ds(g * GB, GB), :] = jnp.sum(c * oh, axis=1, keepdims=True)
        totg_ref[pl.ds(g, 1), :] = jnp.sum(oh, axis=0, keepdims=True)
        return 0

    lax.fori_loop(0, G, g1, 0)

    totg = totg_ref[...]
    sl = (lax.broadcasted_iota(jnp.int32, (G, G), 0)
          > lax.broadcasted_iota(jnp.int32, (G, G), 1)).astype(jnp.float32)
    goff = jnp.dot(sl, totg, preferred_element_type=jnp.float32)
    counts = jnp.sum(totg, axis=0, keepdims=True)
    su = (lax.broadcasted_iota(jnp.int32, (E, E), 0)
          < lax.broadcasted_iota(jnp.int32, (E, E), 1)).astype(jnp.float32)
    offsets = jnp.dot(counts, su, preferred_element_type=jnp.float32)
    totg_ref[...] = offsets + goff

    def g2(g, _):
        oh = oh_ref[pl.ds(g * GB, GB), :]
        base = jnp.sum(oh * totg_ref[pl.ds(g, 1), :],
                       axis=1, keepdims=True)
        rw = rankw_ref[pl.ds(g * GB, GB), :]
        pos_ref[pl.ds(g * GB, GB), :] = (base + rw - 1.0).astype(jnp.int32)
        return 0

    lax.fori_loop(0, G, g2, 0)

    meta_ref[...] = jnp.concatenate(
        [offsets, counts], axis=1).astype(jnp.int32)


def _router(x, wr, br):
    return pl.pallas_call(
        _router_body,
        out_shape=(
            jax.ShapeDtypeStruct((T, 1), jnp.int32),
            jax.ShapeDtypeStruct((T, 128), jnp.float32),
            jax.ShapeDtypeStruct((1, 2 * E), jnp.int32),
        ),
        scratch_shapes=[
            pltpu.VMEM((T, 1), jnp.float32),
            pltpu.VMEM((G, E), jnp.float32),
            pltpu.VMEM((T, E), jnp.float32),
        ],
    )(x, wr, br)


_SC_MESH = plsc.VectorSubcoreMesh(core_axis_name="c", subcore_axis_name="s")


def _dispatch(x, pos, gate16):
    @functools.partial(
        pl.kernel,
        mesh=_SC_MESH,
        out_type=(
            jax.ShapeDtypeStruct((T, D), jnp.float32),
            jax.ShapeDtypeStruct((T, 128), jnp.float32),
        ),
        scratch_types=[
            pltpu.VMEM((WCHUNK,), jnp.int32),
            pltpu.VMEM((WCHUNK, D), jnp.float32),
            pltpu.VMEM((WCHUNK, 128), jnp.float32),
            pltpu.SemaphoreType.DMA,
        ],
    )
    def disp(x_hbm, pos_hbm, g_hbm, xs_hbm, gs_hbm, idx_v, rows_v, g_v, sem):
        wid = lax.axis_index("s") * 2 + lax.axis_index("c")
        base = wid * WCHUNK
        pltpu.sync_copy(pos_hbm.at[pl.ds(base, WCHUNK)], idx_v)
        pltpu.sync_copy(x_hbm.at[pl.ds(base, WCHUNK)], rows_v)
        pltpu.sync_copy(g_hbm.at[pl.ds(base, WCHUNK)], g_v)
        pltpu.async_copy(rows_v, xs_hbm.at[idx_v], sem).wait()
        pltpu.async_copy(g_v, gs_hbm.at[idx_v], sem).wait()

    return disp(x, pos, gate16)


def _unsort(ys, pos):
    @functools.partial(
        pl.kernel,
        mesh=_SC_MESH,
        out_type=jax.ShapeDtypeStruct((T, D), jnp.float32),
        scratch_types=[
            pltpu.VMEM((WCHUNK,), jnp.int32),
            pltpu.VMEM((WCHUNK, D), jnp.float32),
            pltpu.SemaphoreType.DMA,
        ],
    )
    def gath(ys_hbm, pos_hbm, out_hbm, idx_v, rows_v, sem):
        wid = lax.axis_index("s") * 2 + lax.axis_index("c")
        base = wid * WCHUNK
        pltpu.sync_copy(pos_hbm.at[pl.ds(base, WCHUNK)], idx_v)
        pltpu.async_copy(ys_hbm.at[idx_v], rows_v, sem).wait()
        pltpu.sync_copy(rows_v, out_hbm.at[pl.ds(base, WCHUNK)])

    return gath(ys, pos)


def _gemm_body(meta_ref, xs_ref, gs_ref, w1_ref, b1_ref, w2_ref, b2_ref,
               w3_ref, b3_ref, out_ref):
    e = pl.program_id(0)
    off = meta_ref[e]
    cnt = meta_ref[E + e]
    w1 = w1_ref[0].astype(jnp.bfloat16)
    w2 = w2_ref[0].astype(jnp.bfloat16)
    w3 = w3_ref[0].astype(jnp.bfloat16)
    b1 = b1_ref[0]
    b2 = b2_ref[0]
    b3 = b3_ref[0]
    off8 = (off // 8) * 8
    d = off - off8
    nt = jnp.where(cnt > 0, (cnt + d + BLK - 1) // BLK, 0)

    def chunk(i, _):
        start = pl.multiple_of(jnp.minimum(off8 + i * BLK, T - BLK), 8)
        rows = xs_ref[pl.ds(start, BLK), :].astype(jnp.bfloat16)
        h = jnp.maximum(
            jnp.dot(rows, w1, preferred_element_type=jnp.float32) + b1, 0.0)
        h = jnp.maximum(
            jnp.dot(h.astype(jnp.bfloat16), w2,
                    preferred_element_type=jnp.float32) + b2, 0.0)
        o = jnp.dot(h.astype(jnp.bfloat16), w3,
                    preferred_element_type=jnp.float32) + b3
        o = o * gs_ref[pl.ds(start, BLK), 0:1]
        rowid = start + lax.broadcasted_iota(jnp.int32, (BLK, 1), 0)
        mask = (rowid >= off) & (rowid < off + cnt)
        cur = out_ref[pl.ds(start, BLK), :]
        out_ref[pl.ds(start, BLK), :] = jnp.where(mask, o, cur)
        return 0

    lax.fori_loop(0, nt, chunk, 0)


def _gemm(meta, xs, gs, w1, b1, w2, b2, w3, b3):
    grid_spec = pltpu.PrefetchScalarGridSpec(
        num_scalar_prefetch=1,
        grid=(E,),
        in_specs=[
            pl.BlockSpec((T, D), lambda e, m: (0, 0)),
            pl.BlockSpec((T, 128), lambda e, m: (0, 0)),
            pl.BlockSpec((1, D, H), lambda e, m: (e, 0, 0)),
            pl.BlockSpec((1, 1, H), lambda e, m: (e, 0, 0)),
            pl.BlockSpec((1, H, H), lambda e, m: (e, 0, 0)),
            pl.BlockSpec((1, 1, H), lambda e, m: (e, 0, 0)),
            pl.BlockSpec((1, H, D), lambda e, m: (e, 0, 0)),
            pl.BlockSpec((1, 1, D), lambda e, m: (e, 0, 0)),
        ],
        out_specs=pl.BlockSpec((T, D), lambda e, m: (0, 0)),
    )
    return pl.pallas_call(
        _gemm_body,
        grid_spec=grid_spec,
        out_shape=jax.ShapeDtypeStruct((T, D), jnp.float32),
        compiler_params=pltpu.CompilerParams(
            dimension_semantics=("arbitrary",),
        ),
    )(meta, xs, gs, w1, b1, w2, b2, w3, b3)


def kernel(x, Wr, br, W1, b1, W2, b2, W3, b3):
    pos2d, gate16, meta2d = _router(x, Wr, br)
    pos = pos2d.reshape(T)
    meta = meta2d.reshape(2 * E)
    xs, gs = _dispatch(x, pos, gate16)
    ys = _gemm(meta, xs, gs,
               W1, b1.reshape(E, 1, H), W2, b2.reshape(E, 1, H),
               W3, b3.reshape(E, 1, D))
    return _unsort(ys, pos)


# speedup vs baseline: 10.8004x; 1.0130x over previous
import functools

import jax
import jax.numpy as jnp
from jax import lax
from jax.experimental import pallas as pl
from jax.experimental.pallas import tpu as pltpu
from jax.experimental.pallas import tpu_sc as plsc

T = 4096
D = 768
H = 768
E = 64
G = 8
GB = T // G
BLK = 128
NW = 32
WCHUNK = T // NW


def _router_body(x_ref, wr_ref, br_ref, pos_ref, gate16_ref, meta_ref,
                 rankw_ref, totg_ref, oh_ref):
    x = x_ref[...]
    logits = jnp.dot(x, wr_ref[...], preferred_element_type=jnp.float32)
    logits = logits + br_ref[...].reshape(1, E)
    m = jnp.max(logits, axis=1, keepdims=True)
    el = jnp.exp(logits - m)
    s = jnp.sum(el, axis=1, keepdims=True)
    p = el / s
    pmax = jnp.max(p, axis=1, keepdims=True)
    lane = lax.broadcasted_iota(jnp.int32, (T, E), 1)
    idx = jnp.min(jnp.where(p >= pmax, lane, E), axis=1, keepdims=True)
    oh_ref[...] = (lane == idx).astype(jnp.float32)
    gate = pmax / (pmax + 1e-6)
    gate16_ref[...] = jnp.broadcast_to(gate, (T, 128))

    tri = (lax.broadcasted_iota(jnp.int32, (GB, GB), 0)
           >= lax.broadcasted_iota(jnp.int32, (GB, GB), 1)).astype(jnp.bfloat16)

    def g1(g, _):
        oh = oh_ref[pl.ds(g * GB, GB), :]
        c = jnp.dot(tri, oh.astype(jnp.bfloat16),
                    preferred_element_type=jnp.float32)
        rankw_ref[pl.ds(g * GB, GB), :] = jnp.sum(c * oh, axis=1, keepdims=True)
        totg_ref[pl.ds(g, 1), :] = jnp.sum(oh, axis=0, keepdims=True)
        return 0

    lax.fori_loop(0, G, g1, 0)

    totg = totg_ref[...]
    sl = (lax.broadcasted_iota(jnp.int32, (G, G), 0)
          > lax.broadcasted_iota(jnp.int32, (G, G), 1)).astype(jnp.float32)
    goff = jnp.dot(sl, totg, preferred_element_type=jnp.float32)
    counts = jnp.sum(totg, axis=0, keepdims=True)
    su = (lax.broadcasted_iota(jnp.int32, (E, E), 0)
          < lax.broadcasted_iota(jnp.int32, (E, E), 1)).astype(jnp.float32)
    offsets = jnp.dot(counts, su, preferred_element_type=jnp.float32)
    totg_ref[...] = offsets + goff

    def g2(g, _):
        oh = oh_ref[pl.ds(g * GB, GB), :]
        base = jnp.sum(oh * totg_ref[pl.ds(g, 1), :],
                       axis=1, keepdims=True)
        rw = rankw_ref[pl.ds(g * GB, GB), :]
        pos_ref[pl.ds(g * GB, GB), :] = (base + rw - 1.0).astype(jnp.int32)
        return 0

    lax.fori_loop(0, G, g2, 0)

    meta_ref[...] = jnp.concatenate(
        [offsets, counts], axis=1).astype(jnp.int32)


def _router(x, wr, br):
    return pl.pallas_call(
        _router_body,
        out_shape=(
            jax.ShapeDtypeStruct((T, 1), jnp.int32),
            jax.ShapeDtypeStruct((T, 128), jnp.float32),
            jax.ShapeDtypeStruct((1, 2 * E), jnp.int32),
        ),
        scratch_shapes=[
            pltpu.VMEM((T, 1), jnp.float32),
            pltpu.VMEM((G, E), jnp.float32),
            pltpu.VMEM((T, E), jnp.float32),
        ],
    )(x, wr, br)


_SC_MESH = plsc.VectorSubcoreMesh(core_axis_name="c", subcore_axis_name="s")


def _dispatch(x, pos, gate16):
    @functools.partial(
        pl.kernel,
        mesh=_SC_MESH,
        out_type=(
            jax.ShapeDtypeStruct((T, D), jnp.float32),
            jax.ShapeDtypeStruct((T, 128), jnp.float32),
        ),
        scratch_types=[
            pltpu.VMEM((WCHUNK,), jnp.int32),
            pltpu.VMEM((WCHUNK, D), jnp.float32),
            pltpu.VMEM((WCHUNK, 128), jnp.float32),
            pltpu.SemaphoreType.DMA,
        ],
    )
    def disp(x_hbm, pos_hbm, g_hbm, xs_hbm, gs_hbm, idx_v, rows_v, g_v, sem):
        wid = lax.axis_index("s") * 2 + lax.axis_index("c")
        base = wid * WCHUNK
        pltpu.sync_copy(pos_hbm.at[pl.ds(base, WCHUNK)], idx_v)
        pltpu.sync_copy(x_hbm.at[pl.ds(base, WCHUNK)], rows_v)
        pltpu.sync_copy(g_hbm.at[pl.ds(base, WCHUNK)], g_v)
        pltpu.async_copy(rows_v, xs_hbm.at[idx_v], sem).wait()
        pltpu.async_copy(g_v, gs_hbm.at[idx_v], sem).wait()

    return disp(x, pos, gate16)


def _unsort(ys, pos):
    @functools.partial(
        pl.kernel,
        mesh=_SC_MESH,
        out_type=jax.ShapeDtypeStruct((T, D), jnp.float32),
        scratch_types=[
            pltpu.VMEM((WCHUNK,), jnp.int32),
            pltpu.VMEM((WCHUNK, D), jnp.float32),
            pltpu.SemaphoreType.DMA,
        ],
    )
    def gath(ys_hbm, pos_hbm, out_hbm, idx_v, rows_v, sem):
        wid = lax.axis_index("s") * 2 + lax.axis_index("c")
        base = wid * WCHUNK
        pltpu.sync_copy(pos_hbm.at[pl.ds(base, WCHUNK)], idx_v)
        pltpu.async_copy(ys_hbm.at[idx_v], rows_v, sem).wait()
        pltpu.sync_copy(rows_v, out_hbm.at[pl.ds(base, WCHUNK)])

    return gath(ys, pos)


def _gemm_body(meta_ref, xs_ref, gs_ref, w1_ref, b1_ref, w2_ref, b2_ref,
               w3_ref, b3_ref, out_ref):
    e = pl.program_id(0)
    off = meta_ref[e]
    cnt = meta_ref[E + e]
    w1 = w1_ref[0].astype(jnp.bfloat16)
    w2 = w2_ref[0].astype(jnp.bfloat16)
    w3 = w3_ref[0].astype(jnp.bfloat16)
    b1 = b1_ref[0]
    b2 = b2_ref[0]
    b3 = b3_ref[0]
    off8 = (off // 8) * 8
    d = off - off8
    nt = jnp.where(cnt > 0, (cnt + d + BLK - 1) // BLK, 0)

    def chunk(i, _):
        start = pl.multiple_of(jnp.minimum(off8 + i * BLK, T - BLK), 8)
        rows = xs_ref[pl.ds(start, BLK), :].astype(jnp.bfloat16)
        h = jnp.maximum(
            jnp.dot(rows, w1, preferred_element_type=jnp.float32) + b1, 0.0)
        h = jnp.maximum(
            jnp.dot(h.astype(jnp.bfloat16), w2,
                    preferred_element_type=jnp.float32) + b2, 0.0)
        o = jnp.dot(h.astype(jnp.bfloat16), w3,
                    preferred_element_type=jnp.float32) + b3
        o = o * gs_ref[pl.ds(start, BLK), 0:1]
        rowid = start + lax.broadcasted_iota(jnp.int32, (BLK, 1), 0)
        mask = (rowid >= off) & (rowid < off + cnt)
        cur = out_ref[pl.ds(start, BLK), :]
        out_ref[pl.ds(start, BLK), :] = jnp.where(mask, o, cur)
        return 0

    lax.fori_loop(0, nt, chunk, 0)


def _gemm(meta, xs, gs, w1, b1, w2, b2, w3, b3):
    grid_spec = pltpu.PrefetchScalarGridSpec(
        num_scalar_prefetch=1,
        grid=(E,),
        in_specs=[
            pl.BlockSpec((T, D), lambda e, m: (0, 0)),
            pl.BlockSpec((T, 128), lambda e, m: (0, 0)),
            pl.BlockSpec((1, D, H), lambda e, m: (e, 0, 0)),
            pl.BlockSpec((1, 1, H), lambda e, m: (e, 0, 0)),
            pl.BlockSpec((1, H, H), lambda e, m: (e, 0, 0)),
            pl.BlockSpec((1, 1, H), lambda e, m: (e, 0, 0)),
            pl.BlockSpec((1, H, D), lambda e, m: (e, 0, 0)),
            pl.BlockSpec((1, 1, D), lambda e, m: (e, 0, 0)),
        ],
        out_specs=pl.BlockSpec((T, D), lambda e, m: (0, 0)),
    )
    return pl.pallas_call(
        _gemm_body,
        grid_spec=grid_spec,
        out_shape=jax.ShapeDtypeStruct((T, D), jnp.float32),
        compiler_params=pltpu.CompilerParams(
            dimension_semantics=("arbitrary",),
        ),
    )(meta, xs, gs, w1, b1, w2, b2, w3, b3)


def kernel(x, Wr, br, W1, b1, W2, b2, W3, b3):
    pos2d, gate16, meta2d = _router(x, Wr, br)
    pos = pos2d.reshape(T)
    meta = meta2d.reshape(2 * E)
    xs, gs = _dispatch(x, pos, gate16)
    ys = _gemm(meta, xs, gs,
               W1, b1.reshape(E, 1, H), W2, b2.reshape(E, 1, H),
               W3, b3.reshape(E, 1, D))
    return _unsort(ys, pos)


# speedup vs baseline: 10.9600x; 1.0148x over previous
import functools

import jax
import jax.numpy as jnp
from jax import lax
from jax.experimental import pallas as pl
from jax.experimental.pallas import tpu as pltpu
from jax.experimental.pallas import tpu_sc as plsc

T = 4096
D = 768
H = 768
E = 64
G = 8
GB = T // G
BLK = 128
NW = 32
WCHUNK = T // NW


def _router_body(x_ref, wr_ref, br_ref, pos_ref, gate16_ref, meta_ref,
                 rankw_ref, totg_ref, oh_ref):
    x = x_ref[...]
    logits = jnp.dot(x, wr_ref[...], preferred_element_type=jnp.float32)
    logits = logits + br_ref[...].reshape(1, E)
    m = jnp.max(logits, axis=1, keepdims=True)
    el = jnp.exp(logits - m)
    s = jnp.sum(el, axis=1, keepdims=True)
    lane = lax.broadcasted_iota(jnp.int32, (T, E), 1)
    idx = jnp.min(jnp.where(el >= 1.0, lane, E), axis=1, keepdims=True)
    oh_ref[...] = (lane == idx).astype(jnp.float32)
    gate = 1.0 / (1.0 + 1e-6 * s)
    gate16_ref[...] = jnp.broadcast_to(gate, (T, 128))

    tri = (lax.broadcasted_iota(jnp.int32, (GB, GB), 0)
           >= lax.broadcasted_iota(jnp.int32, (GB, GB), 1)).astype(jnp.bfloat16)

    def g1(g, _):
        oh = oh_ref[pl.ds(g * GB, GB), :]
        c = jnp.dot(tri, oh.astype(jnp.bfloat16),
                    preferred_element_type=jnp.float32)
        rankw_ref[pl.ds(g * GB, GB), :] = jnp.sum(c * oh, axis=1, keepdims=True)
        totg_ref[pl.ds(g, 1), :] = jnp.sum(oh, axis=0, keepdims=True)
        return 0

    lax.fori_loop(0, G, g1, 0)

    totg = totg_ref[...]
    sl = (lax.broadcasted_iota(jnp.int32, (G, G), 0)
          > lax.broadcasted_iota(jnp.int32, (G, G), 1)).astype(jnp.float32)
    goff = jnp.dot(sl, totg, preferred_element_type=jnp.float32)
    counts = jnp.sum(totg, axis=0, keepdims=True)
    su = (lax.broadcasted_iota(jnp.int32, (E, E), 0)
          < lax.broadcasted_iota(jnp.int32, (E, E), 1)).astype(jnp.float32)
    offsets = jnp.dot(counts, su, preferred_element_type=jnp.float32)
    totg_ref[...] = offsets + goff

    def g2(g, _):
        oh = oh_ref[pl.ds(g * GB, GB), :]
        base = jnp.sum(oh * totg_ref[pl.ds(g, 1), :],
                       axis=1, keepdims=True)
        rw = rankw_ref[pl.ds(g * GB, GB), :]
        pos_ref[pl.ds(g * GB, GB), :] = (base + rw - 1.0).astype(jnp.int32)
        return 0

    lax.fori_loop(0, G, g2, 0)

    meta_ref[...] = jnp.concatenate(
        [offsets, counts], axis=1).astype(jnp.int32)


def _router(x, wr, br):
    return pl.pallas_call(
        _router_body,
        out_shape=(
            jax.ShapeDtypeStruct((T, 1), jnp.int32),
            jax.ShapeDtypeStruct((T, 128), jnp.float32),
            jax.ShapeDtypeStruct((1, 2 * E), jnp.int32),
        ),
        scratch_shapes=[
            pltpu.VMEM((T, 1), jnp.float32),
            pltpu.VMEM((G, E), jnp.float32),
            pltpu.VMEM((T, E), jnp.float32),
        ],
    )(x, wr, br)


_SC_MESH = plsc.VectorSubcoreMesh(core_axis_name="c", subcore_axis_name="s")


HW = WCHUNK // 2


def _dispatch(x, pos, gate16):
    @functools.partial(
        pl.kernel,
        mesh=_SC_MESH,
        out_type=(
            jax.ShapeDtypeStruct((T, D), jnp.float32),
            jax.ShapeDtypeStruct((T, 128), jnp.float32),
        ),
        scratch_types=[
            pltpu.VMEM((HW,), jnp.int32),
            pltpu.VMEM((HW,), jnp.int32),
            pltpu.VMEM((HW, D), jnp.float32),
            pltpu.VMEM((HW, D), jnp.float32),
            pltpu.VMEM((HW, 128), jnp.float32),
            pltpu.VMEM((HW, 128), jnp.float32),
            pltpu.SemaphoreType.DMA,
            pltpu.SemaphoreType.DMA,
            pltpu.SemaphoreType.DMA,
            pltpu.SemaphoreType.DMA,
            pltpu.SemaphoreType.DMA,
            pltpu.SemaphoreType.DMA,
            pltpu.SemaphoreType.DMA,
            pltpu.SemaphoreType.DMA,
        ],
    )
    def disp(x_hbm, pos_hbm, g_hbm, xs_hbm, gs_hbm,
             i0, i1, x0, x1, g0, g1, sx0, sx1, sg0, sg1, ox0, ox1, og0, og1):
        wid = lax.axis_index("s") * 2 + lax.axis_index("c")
        base = wid * WCHUNK
        pltpu.sync_copy(pos_hbm.at[0, pl.ds(base, HW)], i0)
        pltpu.sync_copy(pos_hbm.at[0, pl.ds(base + HW, HW)], i1)
        ldx0 = pltpu.async_copy(x_hbm.at[pl.ds(base, HW)], x0, sx0)
        ldx1 = pltpu.async_copy(x_hbm.at[pl.ds(base + HW, HW)], x1, sx1)
        ldg0 = pltpu.async_copy(g_hbm.at[pl.ds(base, HW)], g0, sg0)
        ldg1 = pltpu.async_copy(g_hbm.at[pl.ds(base + HW, HW)], g1, sg1)
        ldx0.wait()
        stx0 = pltpu.async_copy(x0, xs_hbm.at[i0], ox0)
        ldg0.wait()
        stg0 = pltpu.async_copy(g0, gs_hbm.at[i0], og0)
        ldx1.wait()
        stx1 = pltpu.async_copy(x1, xs_hbm.at[i1], ox1)
        ldg1.wait()
        stg1 = pltpu.async_copy(g1, gs_hbm.at[i1], og1)
        stx0.wait()
        stg0.wait()
        stx1.wait()
        stg1.wait()

    return disp(x, pos, gate16)


def _unsort(ys, pos):
    @functools.partial(
        pl.kernel,
        mesh=_SC_MESH,
        out_type=jax.ShapeDtypeStruct((T, D), jnp.float32),
        scratch_types=[
            pltpu.VMEM((HW,), jnp.int32),
            pltpu.VMEM((HW,), jnp.int32),
            pltpu.VMEM((HW, D), jnp.float32),
            pltpu.VMEM((HW, D), jnp.float32),
            pltpu.SemaphoreType.DMA,
            pltpu.SemaphoreType.DMA,
            pltpu.SemaphoreType.DMA,
            pltpu.SemaphoreType.DMA,
        ],
    )
    def gath(ys_hbm, pos_hbm, out_hbm, i0, i1, b0, b1, sy0, sy1, so0, so1):
        wid = lax.axis_index("s") * 2 + lax.axis_index("c")
        base = wid * WCHUNK
        pltpu.sync_copy(pos_hbm.at[0, pl.ds(base, HW)], i0)
        pltpu.sync_copy(pos_hbm.at[0, pl.ds(base + HW, HW)], i1)
        g0 = pltpu.async_copy(ys_hbm.at[i0], b0, sy0)
        g1 = pltpu.async_copy(ys_hbm.at[i1], b1, sy1)
        g0.wait()
        w0 = pltpu.async_copy(b0, out_hbm.at[pl.ds(base, HW)], so0)
        g1.wait()
        w1 = pltpu.async_copy(b1, out_hbm.at[pl.ds(base + HW, HW)], so1)
        w0.wait()
        w1.wait()

    return gath(ys, pos)


def _gemm_body(meta_ref, xs_ref, gs_ref, w1_ref, b1_ref, w2_ref, b2_ref,
               w3_ref, b3_ref, out_ref):
    e = pl.program_id(0)
    off = meta_ref[e]
    cnt = meta_ref[E + e]
    w1 = w1_ref[0].astype(jnp.bfloat16)
    w2 = w2_ref[0].astype(jnp.bfloat16)
    w3 = w3_ref[0].astype(jnp.bfloat16)
    b1 = b1_ref[0]
    b2 = b2_ref[0]
    b3 = b3_ref[0]
    off8 = (off // 8) * 8
    d = off - off8
    nt = jnp.where(cnt > 0, (cnt + d + BLK - 1) // BLK, 0)

    def chunk(i, _):
        start = pl.multiple_of(jnp.minimum(off8 + i * BLK, T - BLK), 8)
        rows = xs_ref[pl.ds(start, BLK), :].astype(jnp.bfloat16)
        h = jnp.maximum(
            jnp.dot(rows, w1, preferred_element_type=jnp.float32) + b1, 0.0)
        h = jnp.maximum(
            jnp.dot(h.astype(jnp.bfloat16), w2,
                    preferred_element_type=jnp.float32) + b2, 0.0)
        o = jnp.dot(h.astype(jnp.bfloat16), w3,
                    preferred_element_type=jnp.float32) + b3
        o = o * gs_ref[pl.ds(start, BLK), 0:1]
        rowid = start + lax.broadcasted_iota(jnp.int32, (BLK, 1), 0)
        mask = (rowid >= off) & (rowid < off + cnt)
        cur = out_ref[pl.ds(start, BLK), :]
        out_ref[pl.ds(start, BLK), :] = jnp.where(mask, o, cur)
        return 0

    lax.fori_loop(0, nt, chunk, 0)


def _gemm(meta, xs, gs, w1, b1, w2, b2, w3, b3):
    grid_spec = pltpu.PrefetchScalarGridSpec(
        num_scalar_prefetch=1,
        grid=(E,),
        in_specs=[
            pl.BlockSpec((T, D), lambda e, m: (0, 0)),
            pl.BlockSpec((T, 128), lambda e, m: (0, 0)),
            pl.BlockSpec((1, D, H), lambda e, m: (e, 0, 0)),
            pl.BlockSpec((1, 1, H), lambda e, m: (e, 0, 0)),
            pl.BlockSpec((1, H, H), lambda e, m: (e, 0, 0)),
            pl.BlockSpec((1, 1, H), lambda e, m: (e, 0, 0)),
            pl.BlockSpec((1, H, D), lambda e, m: (e, 0, 0)),
            pl.BlockSpec((1, 1, D), lambda e, m: (e, 0, 0)),
        ],
        out_specs=pl.BlockSpec((T, D), lambda e, m: (0, 0)),
    )
    return pl.pallas_call(
        _gemm_body,
        grid_spec=grid_spec,
        out_shape=jax.ShapeDtypeStruct((T, D), jnp.float32),
        compiler_params=pltpu.CompilerParams(
            dimension_semantics=("arbitrary",),
        ),
    )(meta, xs, gs, w1, b1, w2, b2, w3, b3)


def kernel(x, Wr, br, W1, b1, W2, b2, W3, b3):
    pos2d, gate16, meta2d = _router(x, Wr, br)
    pos = pos2d.reshape(1, T)
    meta = meta2d.reshape(2 * E)
    xs, gs = _dispatch(x, pos, gate16)
    ys = _gemm(meta, xs, gs,
               W1, b1.reshape(E, 1, H), W2, b2.reshape(E, 1, H),
               W3, b3.reshape(E, 1, D))
    return _unsort(ys, pos)


# speedup vs baseline: 11.2795x; 1.0292x over previous
import functools

import jax
import jax.numpy as jnp
from jax import lax
from jax.experimental import pallas as pl
from jax.experimental.pallas import tpu as pltpu
from jax.experimental.pallas import tpu_sc as plsc

T = 4096
D = 768
H = 768
E = 64
G = 8
GB = T // G
BLK = 128
NW = 32
WCHUNK = T // NW


def _router_body(x_ref, wr_ref, br_ref, pos_ref, gate16_ref, meta_ref,
                 rankw_ref, totg_ref, oh_ref):
    x = x_ref[...]
    logits = jnp.dot(x, wr_ref[...], preferred_element_type=jnp.float32)
    logits = logits + br_ref[...].reshape(1, E)
    m = jnp.max(logits, axis=1, keepdims=True)
    el = jnp.exp(logits - m)
    s = jnp.sum(el, axis=1, keepdims=True)
    lane = lax.broadcasted_iota(jnp.int32, (T, E), 1)
    idx = jnp.min(jnp.where(el >= 1.0, lane, E), axis=1, keepdims=True)
    oh_ref[...] = (lane == idx).astype(jnp.float32)
    gate = 1.0 / (1.0 + 1e-6 * s)
    gate16_ref[...] = jnp.broadcast_to(gate, (T, 128))

    tri = (lax.broadcasted_iota(jnp.int32, (GB, GB), 0)
           >= lax.broadcasted_iota(jnp.int32, (GB, GB), 1)).astype(jnp.bfloat16)

    def g1(g, _):
        oh = oh_ref[pl.ds(g * GB, GB), :]
        c = jnp.dot(tri, oh.astype(jnp.bfloat16),
                    preferred_element_type=jnp.float32)
        rankw_ref[pl.ds(g * GB, GB), :] = jnp.sum(c * oh, axis=1, keepdims=True)
        totg_ref[pl.ds(g, 1), :] = jnp.sum(oh, axis=0, keepdims=True)
        return 0

    lax.fori_loop(0, G, g1, 0)

    totg = totg_ref[...]
    sl = (lax.broadcasted_iota(jnp.int32, (G, G), 0)
          > lax.broadcasted_iota(jnp.int32, (G, G), 1)).astype(jnp.float32)
    goff = jnp.dot(sl, totg, preferred_element_type=jnp.float32)
    counts = jnp.sum(totg, axis=0, keepdims=True)
    su = (lax.broadcasted_iota(jnp.int32, (E, E), 0)
          < lax.broadcasted_iota(jnp.int32, (E, E), 1)).astype(jnp.float32)
    offsets = jnp.dot(counts, su, preferred_element_type=jnp.float32)
    totg_ref[...] = offsets + goff

    def g2(g, _):
        oh = oh_ref[pl.ds(g * GB, GB), :]
        base = jnp.sum(oh * totg_ref[pl.ds(g, 1), :],
                       axis=1, keepdims=True)
        rw = rankw_ref[pl.ds(g * GB, GB), :]
        pos_ref[pl.ds(g * GB, GB), :] = (base + rw - 1.0).astype(jnp.int32)
        return 0

    lax.fori_loop(0, G, g2, 0)

    meta_ref[...] = jnp.concatenate(
        [offsets, counts], axis=1).astype(jnp.int32)


def _router(x, wr, br):
    return pl.pallas_call(
        _router_body,
        out_shape=(
            jax.ShapeDtypeStruct((T, 1), jnp.int32),
            jax.ShapeDtypeStruct((T, 128), jnp.float32),
            jax.ShapeDtypeStruct((1, 2 * E), jnp.int32),
        ),
        scratch_shapes=[
            pltpu.VMEM((T, 1), jnp.float32),
            pltpu.VMEM((G, E), jnp.float32),
            pltpu.VMEM((T, E), jnp.float32),
        ],
    )(x, wr, br)


_SC_MESH = plsc.VectorSubcoreMesh(core_axis_name="c", subcore_axis_name="s")


HW = WCHUNK // 2


def _dispatch(x, pos, gate16):
    @functools.partial(
        pl.kernel,
        mesh=_SC_MESH,
        out_type=(
            jax.ShapeDtypeStruct((T, D), jnp.float32),
            jax.ShapeDtypeStruct((T, 128), jnp.float32),
        ),
        scratch_types=[
            pltpu.VMEM((HW,), jnp.int32),
            pltpu.VMEM((HW,), jnp.int32),
            pltpu.VMEM((HW, D), jnp.float32),
            pltpu.VMEM((HW, D), jnp.float32),
            pltpu.VMEM((HW, 128), jnp.float32),
            pltpu.VMEM((HW, 128), jnp.float32),
            pltpu.SemaphoreType.DMA,
            pltpu.SemaphoreType.DMA,
            pltpu.SemaphoreType.DMA,
            pltpu.SemaphoreType.DMA,
            pltpu.SemaphoreType.DMA,
            pltpu.SemaphoreType.DMA,
            pltpu.SemaphoreType.DMA,
            pltpu.SemaphoreType.DMA,
        ],
    )
    def disp(x_hbm, pos_hbm, g_hbm, xs_hbm, gs_hbm,
             i0, i1, x0, x1, g0, g1, sx0, sx1, sg0, sg1, ox0, ox1, og0, og1):
        wid = lax.axis_index("s") * 2 + lax.axis_index("c")
        base = wid * WCHUNK
        pltpu.sync_copy(pos_hbm.at[0, pl.ds(base, HW)], i0)
        pltpu.sync_copy(pos_hbm.at[0, pl.ds(base + HW, HW)], i1)
        ldx0 = pltpu.async_copy(x_hbm.at[pl.ds(base, HW)], x0, sx0)
        ldx1 = pltpu.async_copy(x_hbm.at[pl.ds(base + HW, HW)], x1, sx1)
        ldg0 = pltpu.async_copy(g_hbm.at[pl.ds(base, HW)], g0, sg0)
        ldg1 = pltpu.async_copy(g_hbm.at[pl.ds(base + HW, HW)], g1, sg1)
        ldx0.wait()
        stx0 = pltpu.async_copy(x0, xs_hbm.at[i0], ox0)
        ldg0.wait()
        stg0 = pltpu.async_copy(g0, gs_hbm.at[i0], og0)
        ldx1.wait()
        stx1 = pltpu.async_copy(x1, xs_hbm.at[i1], ox1)
        ldg1.wait()
        stg1 = pltpu.async_copy(g1, gs_hbm.at[i1], og1)
        stx0.wait()
        stg0.wait()
        stx1.wait()
        stg1.wait()

    return disp(x, pos, gate16)


def _unsort(ys, pos):
    @functools.partial(
        pl.kernel,
        mesh=_SC_MESH,
        out_type=jax.ShapeDtypeStruct((T, D), jnp.float32),
        scratch_types=[
            pltpu.VMEM((HW,), jnp.int32),
            pltpu.VMEM((HW,), jnp.int32),
            pltpu.VMEM((HW, D), jnp.float32),
            pltpu.VMEM((HW, D), jnp.float32),
            pltpu.SemaphoreType.DMA,
            pltpu.SemaphoreType.DMA,
            pltpu.SemaphoreType.DMA,
            pltpu.SemaphoreType.DMA,
        ],
    )
    def gath(ys_hbm, pos_hbm, out_hbm, i0, i1, b0, b1, sy0, sy1, so0, so1):
        wid = lax.axis_index("s") * 2 + lax.axis_index("c")
        base = wid * WCHUNK
        pltpu.sync_copy(pos_hbm.at[0, pl.ds(base, HW)], i0)
        pltpu.sync_copy(pos_hbm.at[0, pl.ds(base + HW, HW)], i1)
        g0 = pltpu.async_copy(ys_hbm.at[i0], b0, sy0)
        g1 = pltpu.async_copy(ys_hbm.at[i1], b1, sy1)
        g0.wait()
        w0 = pltpu.async_copy(b0, out_hbm.at[pl.ds(base, HW)], so0)
        g1.wait()
        w1 = pltpu.async_copy(b1, out_hbm.at[pl.ds(base + HW, HW)], so1)
        w0.wait()
        w1.wait()

    return gath(ys, pos)


NBUF = 3


def _gemm_body(meta_ref, xs_ref, gs_ref, w1_ref, b1_ref, w2_ref, b2_ref,
               w3_ref, b3_ref, out_ref, w1b, w2b, w3b, s1, s2, s3):
    e = pl.program_id(0)
    off = meta_ref[e]
    cnt = meta_ref[E + e]

    def fetch(eidx, slot):
        pltpu.make_async_copy(w1_ref.at[eidx], w1b.at[slot], s1.at[slot]).start()
        pltpu.make_async_copy(w2_ref.at[eidx], w2b.at[slot], s2.at[slot]).start()
        pltpu.make_async_copy(w3_ref.at[eidx], w3b.at[slot], s3.at[slot]).start()

    @pl.when(e == 0)
    def _prologue():
        for k in range(NBUF):
            fetch(k, k)

    slot = lax.rem(e, NBUF)
    pltpu.make_async_copy(w1_ref.at[e], w1b.at[slot], s1.at[slot]).wait()
    pltpu.make_async_copy(w2_ref.at[e], w2b.at[slot], s2.at[slot]).wait()
    pltpu.make_async_copy(w3_ref.at[e], w3b.at[slot], s3.at[slot]).wait()

    w1 = w1b[slot].astype(jnp.bfloat16)
    w2 = w2b[slot].astype(jnp.bfloat16)
    w3 = w3b[slot].astype(jnp.bfloat16)
    b1 = b1_ref[0]
    b2 = b2_ref[0]
    b3 = b3_ref[0]
    off8 = (off // 8) * 8
    d = off - off8
    nt = jnp.where(cnt > 0, (cnt + d + BLK - 1) // BLK, 0)

    def chunk(i, _):
        start = pl.multiple_of(jnp.minimum(off8 + i * BLK, T - BLK), 8)
        rows = xs_ref[pl.ds(start, BLK), :].astype(jnp.bfloat16)
        h = jnp.maximum(
            jnp.dot(rows, w1, preferred_element_type=jnp.float32) + b1, 0.0)
        h = jnp.maximum(
            jnp.dot(h.astype(jnp.bfloat16), w2,
                    preferred_element_type=jnp.float32) + b2, 0.0)
        o = jnp.dot(h.astype(jnp.bfloat16), w3,
                    preferred_element_type=jnp.float32) + b3
        o = o * gs_ref[pl.ds(start, BLK), 0:1]
        rowid = start + lax.broadcasted_iota(jnp.int32, (BLK, 1), 0)
        mask = (rowid >= off) & (rowid < off + cnt)
        cur = out_ref[pl.ds(start, BLK), :]
        out_ref[pl.ds(start, BLK), :] = jnp.where(mask, o, cur)
        return 0

    lax.fori_loop(0, nt, chunk, 0)

    @pl.when(e + NBUF < E)
    def _prefetch_next():
        fetch(e + NBUF, slot)


def _gemm(meta, xs, gs, w1, b1, w2, b2, w3, b3):
    grid_spec = pltpu.PrefetchScalarGridSpec(
        num_scalar_prefetch=1,
        grid=(E,),
        in_specs=[
            pl.BlockSpec((T, D), lambda e, m: (0, 0)),
            pl.BlockSpec((T, 128), lambda e, m: (0, 0)),
            pl.BlockSpec(memory_space=pl.ANY),
            pl.BlockSpec((1, 1, H), lambda e, m: (e, 0, 0)),
            pl.BlockSpec(memory_space=pl.ANY),
            pl.BlockSpec((1, 1, H), lambda e, m: (e, 0, 0)),
            pl.BlockSpec(memory_space=pl.ANY),
            pl.BlockSpec((1, 1, D), lambda e, m: (e, 0, 0)),
        ],
        out_specs=pl.BlockSpec((T, D), lambda e, m: (0, 0)),
        scratch_shapes=[
            pltpu.VMEM((NBUF, D, H), jnp.float32),
            pltpu.VMEM((NBUF, H, H), jnp.float32),
            pltpu.VMEM((NBUF, H, D), jnp.float32),
            pltpu.SemaphoreType.DMA((NBUF,)),
            pltpu.SemaphoreType.DMA((NBUF,)),
            pltpu.SemaphoreType.DMA((NBUF,)),
        ],
    )
    return pl.pallas_call(
        _gemm_body,
        grid_spec=grid_spec,
        out_shape=jax.ShapeDtypeStruct((T, D), jnp.float32),
        compiler_params=pltpu.CompilerParams(
            dimension_semantics=("arbitrary",),
        ),
    )(meta, xs, gs, w1, b1, w2, b2, w3, b3)


def kernel(x, Wr, br, W1, b1, W2, b2, W3, b3):
    pos2d, gate16, meta2d = _router(x, Wr, br)
    pos = pos2d.reshape(1, T)
    meta = meta2d.reshape(2 * E)
    xs, gs = _dispatch(x, pos, gate16)
    ys = _gemm(meta, xs, gs,
               W1, b1.reshape(E, 1, H), W2, b2.reshape(E, 1, H),
               W3, b3.reshape(E, 1, D))
    return _unsort(ys, pos)


# speedup vs baseline: 11.6630x; 1.0340x over previous
import functools

import jax
import jax.numpy as jnp
from jax import lax
from jax.experimental import pallas as pl
from jax.experimental.pallas import tpu as pltpu
from jax.experimental.pallas import tpu_sc as plsc

T = 4096
D = 768
H = 768
E = 64
G = 8
GB = T // G
BLK = 128
NW = 32
WCHUNK = T // NW


def _router_body(x_ref, wr_ref, br_ref, pos_ref, gate16_ref, meta_ref,
                 rankw_ref, totg_ref, oh_ref):
    x = x_ref[...]
    logits = jnp.dot(x, wr_ref[...], preferred_element_type=jnp.float32)
    logits = logits + br_ref[...].reshape(1, E)
    m = jnp.max(logits, axis=1, keepdims=True)
    el = jnp.exp(logits - m)
    s = jnp.sum(el, axis=1, keepdims=True)
    lane = lax.broadcasted_iota(jnp.int32, (T, E), 1)
    idx = jnp.min(jnp.where(el >= 1.0, lane, E), axis=1, keepdims=True)
    oh_ref[...] = (lane == idx).astype(jnp.float32)
    gate = 1.0 / (1.0 + 1e-6 * s)
    gate16_ref[...] = jnp.broadcast_to(gate, (T, 128))

    tri = (lax.broadcasted_iota(jnp.int32, (GB, GB), 0)
           >= lax.broadcasted_iota(jnp.int32, (GB, GB), 1)).astype(jnp.bfloat16)

    def g1(g, _):
        oh = oh_ref[pl.ds(g * GB, GB), :]
        c = jnp.dot(tri, oh.astype(jnp.bfloat16),
                    preferred_element_type=jnp.float32)
        rankw_ref[pl.ds(g * GB, GB), :] = jnp.sum(c * oh, axis=1, keepdims=True)
        totg_ref[pl.ds(g, 1), :] = jnp.sum(oh, axis=0, keepdims=True)
        return 0

    lax.fori_loop(0, G, g1, 0)

    totg = totg_ref[...]
    sl = (lax.broadcasted_iota(jnp.int32, (G, G), 0)
          > lax.broadcasted_iota(jnp.int32, (G, G), 1)).astype(jnp.float32)
    goff = jnp.dot(sl, totg, preferred_element_type=jnp.float32)
    counts = jnp.sum(totg, axis=0, keepdims=True)
    su = (lax.broadcasted_iota(jnp.int32, (E, E), 0)
          < lax.broadcasted_iota(jnp.int32, (E, E), 1)).astype(jnp.float32)
    offsets = jnp.dot(counts, su, preferred_element_type=jnp.float32)
    combined = offsets + goff

    gh = (lax.broadcasted_iota(jnp.int32, (T, G), 0) // GB
          == lax.broadcasted_iota(jnp.int32, (T, G), 1)).astype(jnp.float32)
    ctok = jnp.dot(gh, combined, preferred_element_type=jnp.float32)
    base = jnp.sum(oh_ref[...] * ctok, axis=1, keepdims=True)
    pos_ref[...] = (base + rankw_ref[...] - 1.0).astype(jnp.int32)

    meta_ref[...] = jnp.concatenate(
        [offsets, counts], axis=1).astype(jnp.int32)


def _router(x, wr, br):
    return pl.pallas_call(
        _router_body,
        out_shape=(
            jax.ShapeDtypeStruct((T, 1), jnp.int32),
            jax.ShapeDtypeStruct((T, 128), jnp.float32),
            jax.ShapeDtypeStruct((1, 2 * E), jnp.int32),
        ),
        scratch_shapes=[
            pltpu.VMEM((T, 1), jnp.float32),
            pltpu.VMEM((G, E), jnp.float32),
            pltpu.VMEM((T, E), jnp.float32),
        ],
    )(x, wr, br)


_SC_MESH = plsc.VectorSubcoreMesh(core_axis_name="c", subcore_axis_name="s")


HW = WCHUNK // 2


def _dispatch(x, pos, gate16):
    @functools.partial(
        pl.kernel,
        mesh=_SC_MESH,
        out_type=(
            jax.ShapeDtypeStruct((T, D), jnp.float32),
            jax.ShapeDtypeStruct((T, 128), jnp.float32),
        ),
        scratch_types=[
            pltpu.VMEM((HW,), jnp.int32),
            pltpu.VMEM((HW,), jnp.int32),
            pltpu.VMEM((HW, D), jnp.float32),
            pltpu.VMEM((HW, D), jnp.float32),
            pltpu.VMEM((HW, 128), jnp.float32),
            pltpu.VMEM((HW, 128), jnp.float32),
            pltpu.SemaphoreType.DMA,
            pltpu.SemaphoreType.DMA,
            pltpu.SemaphoreType.DMA,
            pltpu.SemaphoreType.DMA,
            pltpu.SemaphoreType.DMA,
            pltpu.SemaphoreType.DMA,
            pltpu.SemaphoreType.DMA,
            pltpu.SemaphoreType.DMA,
        ],
    )
    def disp(x_hbm, pos_hbm, g_hbm, xs_hbm, gs_hbm,
             i0, i1, x0, x1, g0, g1, sx0, sx1, sg0, sg1, ox0, ox1, og0, og1):
        wid = lax.axis_index("s") * 2 + lax.axis_index("c")
        base = wid * WCHUNK
        pltpu.sync_copy(pos_hbm.at[0, pl.ds(base, HW)], i0)
        pltpu.sync_copy(pos_hbm.at[0, pl.ds(base + HW, HW)], i1)
        ldx0 = pltpu.async_copy(x_hbm.at[pl.ds(base, HW)], x0, sx0)
        ldx1 = pltpu.async_copy(x_hbm.at[pl.ds(base + HW, HW)], x1, sx1)
        ldg0 = pltpu.async_copy(g_hbm.at[pl.ds(base, HW)], g0, sg0)
        ldg1 = pltpu.async_copy(g_hbm.at[pl.ds(base + HW, HW)], g1, sg1)
        ldx0.wait()
        stx0 = pltpu.async_copy(x0, xs_hbm.at[i0], ox0)
        ldg0.wait()
        stg0 = pltpu.async_copy(g0, gs_hbm.at[i0], og0)
        ldx1.wait()
        stx1 = pltpu.async_copy(x1, xs_hbm.at[i1], ox1)
        ldg1.wait()
        stg1 = pltpu.async_copy(g1, gs_hbm.at[i1], og1)
        stx0.wait()
        stg0.wait()
        stx1.wait()
        stg1.wait()

    return disp(x, pos, gate16)


def _unsort(ys, pos):
    @functools.partial(
        pl.kernel,
        mesh=_SC_MESH,
        out_type=jax.ShapeDtypeStruct((T, D), jnp.float32),
        scratch_types=[
            pltpu.VMEM((HW,), jnp.int32),
            pltpu.VMEM((HW,), jnp.int32),
            pltpu.VMEM((HW, D), jnp.float32),
            pltpu.VMEM((HW, D), jnp.float32),
            pltpu.SemaphoreType.DMA,
            pltpu.SemaphoreType.DMA,
            pltpu.SemaphoreType.DMA,
            pltpu.SemaphoreType.DMA,
        ],
    )
    def gath(ys_hbm, pos_hbm, out_hbm, i0, i1, b0, b1, sy0, sy1, so0, so1):
        wid = lax.axis_index("s") * 2 + lax.axis_index("c")
        base = wid * WCHUNK
        pltpu.sync_copy(pos_hbm.at[0, pl.ds(base, HW)], i0)
        pltpu.sync_copy(pos_hbm.at[0, pl.ds(base + HW, HW)], i1)
        g0 = pltpu.async_copy(ys_hbm.at[i0], b0, sy0)
        g1 = pltpu.async_copy(ys_hbm.at[i1], b1, sy1)
        g0.wait()
        w0 = pltpu.async_copy(b0, out_hbm.at[pl.ds(base, HW)], so0)
        g1.wait()
        w1 = pltpu.async_copy(b1, out_hbm.at[pl.ds(base + HW, HW)], so1)
        w0.wait()
        w1.wait()

    return gath(ys, pos)


NBUF = 3


def _gemm_body(meta_ref, xs_ref, gs_ref, w1_ref, b1_ref, w2_ref, b2_ref,
               w3_ref, b3_ref, out_ref, w1b, w2b, w3b, s1, s2, s3):
    e = pl.program_id(0)
    off = meta_ref[e]
    cnt = meta_ref[E + e]

    def fetch(eidx, slot):
        pltpu.make_async_copy(w1_ref.at[eidx], w1b.at[slot], s1.at[slot]).start()
        pltpu.make_async_copy(w2_ref.at[eidx], w2b.at[slot], s2.at[slot]).start()
        pltpu.make_async_copy(w3_ref.at[eidx], w3b.at[slot], s3.at[slot]).start()

    @pl.when(e == 0)
    def _prologue():
        for k in range(NBUF):
            fetch(k, k)

    slot = lax.rem(e, NBUF)
    pltpu.make_async_copy(w1_ref.at[e], w1b.at[slot], s1.at[slot]).wait()
    pltpu.make_async_copy(w2_ref.at[e], w2b.at[slot], s2.at[slot]).wait()
    pltpu.make_async_copy(w3_ref.at[e], w3b.at[slot], s3.at[slot]).wait()

    w1 = w1b[slot].astype(jnp.bfloat16)
    w2 = w2b[slot].astype(jnp.bfloat16)
    w3 = w3b[slot].astype(jnp.bfloat16)
    b1 = b1_ref[e]
    b2 = b2_ref[e]
    b3 = b3_ref[e]
    off8 = (off // 8) * 8
    d = off - off8
    nt = jnp.where(cnt > 0, (cnt + d + BLK - 1) // BLK, 0)

    def chunk(i, _):
        start = pl.multiple_of(jnp.minimum(off8 + i * BLK, T - BLK), 8)
        rows = xs_ref[pl.ds(start, BLK), :].astype(jnp.bfloat16)
        h = jnp.maximum(
            jnp.dot(rows, w1, preferred_element_type=jnp.float32) + b1, 0.0)
        h = jnp.maximum(
            jnp.dot(h.astype(jnp.bfloat16), w2,
                    preferred_element_type=jnp.float32) + b2, 0.0)
        o = jnp.dot(h.astype(jnp.bfloat16), w3,
                    preferred_element_type=jnp.float32) + b3
        o = o * gs_ref[pl.ds(start, BLK), 0:1]
        rowid = start + lax.broadcasted_iota(jnp.int32, (BLK, 1), 0)
        mask = (rowid >= off) & (rowid < off + cnt)
        cur = out_ref[pl.ds(start, BLK), :]
        out_ref[pl.ds(start, BLK), :] = jnp.where(mask, o, cur)
        return 0

    lax.fori_loop(0, nt, chunk, 0)

    @pl.when(e + NBUF < E)
    def _prefetch_next():
        fetch(e + NBUF, slot)


def _gemm(meta, xs, gs, w1, b1, w2, b2, w3, b3):
    grid_spec = pltpu.PrefetchScalarGridSpec(
        num_scalar_prefetch=1,
        grid=(E,),
        in_specs=[
            pl.BlockSpec((T, D), lambda e, m: (0, 0)),
            pl.BlockSpec((T, 128), lambda e, m: (0, 0)),
            pl.BlockSpec(memory_space=pl.ANY),
            pl.BlockSpec((E, 1, H), lambda e, m: (0, 0, 0)),
            pl.BlockSpec(memory_space=pl.ANY),
            pl.BlockSpec((E, 1, H), lambda e, m: (0, 0, 0)),
            pl.BlockSpec(memory_space=pl.ANY),
            pl.BlockSpec((E, 1, D), lambda e, m: (0, 0, 0)),
        ],
        out_specs=pl.BlockSpec((T, D), lambda e, m: (0, 0)),
        scratch_shapes=[
            pltpu.VMEM((NBUF, D, H), jnp.float32),
            pltpu.VMEM((NBUF, H, H), jnp.float32),
            pltpu.VMEM((NBUF, H, D), jnp.float32),
            pltpu.SemaphoreType.DMA((NBUF,)),
            pltpu.SemaphoreType.DMA((NBUF,)),
            pltpu.SemaphoreType.DMA((NBUF,)),
        ],
    )
    return pl.pallas_call(
        _gemm_body,
        grid_spec=grid_spec,
        out_shape=jax.ShapeDtypeStruct((T, D), jnp.float32),
        compiler_params=pltpu.CompilerParams(
            dimension_semantics=("arbitrary",),
        ),
    )(meta, xs, gs, w1, b1, w2, b2, w3, b3)


def kernel(x, Wr, br, W1, b1, W2, b2, W3, b3):
    pos2d, gate16, meta2d = _router(x, Wr, br)
    pos = pos2d.reshape(1, T)
    meta = meta2d.reshape(2 * E)
    xs, gs = _dispatch(x, pos, gate16)
    ys = _gemm(meta, xs, gs,
               W1, b1.reshape(E, 1, H), W2, b2.reshape(E, 1, H),
               W3, b3.reshape(E, 1, D))
    return _unsort(ys, pos)
